# Initial kernel scaffold; baseline (speedup 1.0000x reference)
import jax
import jax.numpy as jnp
from jax.experimental import pallas as pl


def kernel(pred_base, user_seq, user_seq_len, poi_region_id, fusion_batch_users_embs, fusion_pois_embs, region_emb, alpha):
    raise NotImplementedError("write your pallas kernel here")


# speedup vs baseline: 9.4453x; 9.4453x over previous
import functools

import jax
import jax.numpy as jnp
from jax import lax
from jax.experimental import pallas as pl
from jax.experimental.pallas import tpu as pltpu
from jax.experimental.pallas import tpu_sc as plsc

B = 1024
S = 200
NPOI = 100000
NREG = 1000
D = 64
RECENT_K = 10
TOPM = 50

TB = 8
WCOL = 16384
NCT = (NPOI + WCOL - 1) // WCOL
GBLK = 128
NBM = NCT * (WCOL // GBLK)
LAST_BID = (NPOI - 1) // GBLK
LAST_W = NPOI - LAST_BID * GBLK
NSEL = 64
NW = 32
RPW = B // NW
GPW = (B // 8) // NW
NEGINF = float("-inf")


def _copy_blockmax_body(x_ref, out_ref, bm_ref):
    j = pl.program_id(1)
    x = x_ref[...]
    out_ref[...] = x
    col = j * WCOL + lax.broadcasted_iota(jnp.int32, (TB, WCOL), 1)
    xm = jnp.where(col < NPOI, x, NEGINF)
    bm_ref[...] = jnp.max(xm.reshape(TB, WCOL // GBLK, GBLK), axis=2)


@jax.jit
def _k1(pred):
    return pl.pallas_call(
        _copy_blockmax_body,
        grid=(B // TB, NCT),
        in_specs=[pl.BlockSpec((TB, WCOL), lambda i, j: (i, j))],
        out_specs=[
            pl.BlockSpec((TB, WCOL), lambda i, j: (i, j)),
            pl.BlockSpec((TB, WCOL // GBLK), lambda i, j: (i, j)),
        ],
        out_shape=[
            jax.ShapeDtypeStruct((B, NPOI), jnp.float32),
            jax.ShapeDtypeStruct((B, NBM), jnp.float32),
        ],
    )(pred)


def _thresh_body(alpha_ref, bm_ref, up_ref, re_ref, tau_ref, sc_ref):
    bmv = bm_ref[...]
    bits = lax.bitcast_convert_type(bmv, jnp.int32)
    key = bits ^ jnp.where(bits < 0, jnp.int32(0x7FFFFFFF), jnp.int32(0))

    t0 = jnp.full((TB, 1), jnp.int32(-(2**31)))

    def it(k, t):
        c = t + lax.shift_left(jnp.int32(1), jnp.int32(31) - k)
        cnt = jnp.sum((key >= c).astype(jnp.int32), axis=1, keepdims=True)
        return jnp.where(cnt >= TOPM, c, t)

    t = lax.fori_loop(0, 32, it, t0)
    fb = t ^ jnp.where(t < 0, jnp.int32(0x7FFFFFFF), jnp.int32(0))
    tau = lax.bitcast_convert_type(fb, jnp.float32)
    tau_ref[...] = jnp.broadcast_to(tau, (TB, 128))

    up = up_ref[...]
    sc = lax.dot_general(up, re_ref[...], (((1,), (1,)), ((), ())),
                         preferred_element_type=jnp.float32)
    sc = sc * alpha_ref[...][0:TB, 0:1]
    sc_ref[...] = jnp.concatenate(
        [sc, jnp.zeros((TB, 1024 - NREG), jnp.float32)], axis=1)


@jax.jit
def _k2(alpha_arr, bm, up, re):
    return pl.pallas_call(
        _thresh_body,
        grid=(B // TB,),
        in_specs=[
            pl.BlockSpec((8, 128), lambda i: (0, 0)),
            pl.BlockSpec((TB, NBM), lambda i: (i, 0)),
            pl.BlockSpec((TB, D), lambda i: (i, 0)),
            pl.BlockSpec((NREG, D), lambda i: (0, 0)),
        ],
        out_specs=[
            pl.BlockSpec((TB, 128), lambda i: (i, 0)),
            pl.BlockSpec((TB, 1024), lambda i: (i, 0)),
        ],
        out_shape=[
            jax.ShapeDtypeStruct((B, 128), jnp.float32),
            jax.ShapeDtypeStruct((B, 1024), jnp.float32),
        ],
    )(alpha_arr, bm, up, re)


def _mesh():
    return plsc.VectorSubcoreMesh(core_axis_name="c", subcore_axis_name="s")


def _user_pref_body(seq_hbm, len_hbm, pr_hbm, re_hbm, up_hbm,
                    seqv, lenv, ridv, embv, outv, sem0, sem1):
    cc = lax.axis_index("c")
    ss = lax.axis_index("s")
    wid = ss * 2 + cc
    r0 = wid * RPW
    pltpu.sync_copy(seq_hbm.at[pl.ds(r0, RPW)], seqv)
    pltpu.sync_copy(len_hbm.at[pl.ds(r0, RPW)], lenv)
    k16 = lax.iota(jnp.int32, 16)

    def row(i, carry):
        iv = jnp.full((16,), i, jnp.int32)
        ln = jnp.max(plsc.load_gather(lenv, [iv]))
        start = jnp.maximum(ln - RECENT_K, 0)
        cnt = ln - start
        pos = jnp.minimum(start + k16, S - 1)
        pid = plsc.load_gather(seqv, [iv, pos])
        pltpu.async_copy(pr_hbm.at[pid], ridv, sem0).wait()
        pltpu.async_copy(re_hbm.at[ridv], embv, sem1).wait()
        w = jnp.where(k16 < cnt, jnp.float32(1.0), jnp.float32(0.0))
        inv = jnp.float32(1.0)
        for n in range(2, RECENT_K + 1):
            inv = jnp.where(cnt == n, jnp.float32(1.0 / n), inv)
        for jj in range(D // 16):
            acc = jnp.zeros((16,), jnp.float32)
            for tt in range(16):
                wt = jnp.max(jnp.where(k16 == tt, w, jnp.float32(0.0)))
                acc = acc + embv[tt, pl.ds(16 * jj, 16)] * wt
            plsc.store_scatter(outv, [iv, 16 * jj + k16], acc * inv)
        return carry

    lax.fori_loop(0, RPW, row, 0)
    pltpu.sync_copy(outv, up_hbm.at[pl.ds(r0, RPW)])


@jax.jit
def _s1(user_seq, user_seq_len, poi_region_id, region_emb):
    region_emb = jnp.pad(region_emb, ((0, 0), (0, 128 - D)))
    fn = pl.kernel(
        _user_pref_body,
        mesh=_mesh(),
        compiler_params=pltpu.CompilerParams(needs_layout_passes=False),
        out_type=jax.ShapeDtypeStruct((B, D), jnp.float32),
        scratch_types=[
            pltpu.VMEM((RPW, S), jnp.int32),
            pltpu.VMEM((RPW,), jnp.int32),
            pltpu.VMEM((16,), jnp.int32),
            pltpu.VMEM((16, 128), jnp.float32),
            pltpu.VMEM((RPW, D), jnp.float32),
            pltpu.SemaphoreType.DMA,
            pltpu.SemaphoreType.DMA,
        ],
    )
    return fn(user_seq, user_seq_len, poi_region_id, region_emb)


def _apply_body(bm_hbm, tau_hbm, sc_hbm, pr_hbm, out_ref, tail_ref,
                bm8, tau8, sc8, tailbuf, idsv, stag, selidx, selloc,
                selval, ridv, gsem, ssem, osem):
    cc = lax.axis_index("c")
    ss = lax.axis_index("s")
    wid = ss * 2 + cc
    k16 = lax.iota(jnp.int32, 16)
    lane0 = k16 == 0

    def splat_i(x):
        return jnp.full((16,), x, jnp.int32)

    def splat_f(x):
        return jnp.full((16,), x, jnp.float32)

    def bid_of(k):
        return jnp.max(plsc.load_gather(idsv, [splat_i(k)]))

    def blk_max(k, bid, i8):
        m = splat_f(NEGINF)
        for ii in range(8):
            cv = plsc.load_gather(stag, [splat_i(k), splat_i(i8),
                                         16 * ii + k16])
            ok = bid * GBLK + 16 * ii + k16 < NPOI
            m = jnp.maximum(m, jnp.where(ok, cv, splat_f(NEGINF)))
        return jnp.max(m)

    def group(gl, carry):
        g = wid * GPW + gl
        rg = g * 8
        pltpu.sync_copy(bm_hbm.at[pl.ds(rg, 8)], bm8)
        pltpu.sync_copy(tau_hbm.at[pl.ds(rg, 8)], tau8)
        pltpu.sync_copy(sc_hbm.at[pl.ds(rg, 8)], sc8)

        def row(i8, zz):
            tau = jnp.max(plsc.load_gather(tau8, [splat_i(i8), k16]))

            for j5 in range(5):
                idsv[pl.ds(16 * j5, 16)] = k16 + 16 * j5
            cnt = jnp.int32(0)
            for j in range(NBM // 16):
                v = plsc.load_gather(bm8, [splat_i(i8), 16 * j + k16])
                m = v >= tau
                c_now = cnt

                @pl.when(c_now < NSEL)
                def _():
                    plsc.store_compressed(idsv.at[pl.ds(c_now, 16)],
                                          k16 + 16 * j, mask=m)

                cnt = cnt + jnp.max(plsc.all_reduce_population_count(m))

            ncand = jnp.minimum(cnt, NSEL)

            def gfire(k, z):
                bid = bid_of(k)

                @pl.when(jnp.logical_and(k < ncand, bid < LAST_BID))
                def _():
                    pltpu.async_copy(
                        out_ref.at[pl.ds(rg, 8), pl.ds(bid * GBLK, GBLK)],
                        stag.at[k], gsem)

                @pl.when(jnp.logical_and(k < ncand, bid >= LAST_BID))
                def _():
                    pltpu.sync_copy(tail_ref.at[pl.ds(rg, 8)], tailbuf)
                    for rr in range(8):
                        for ii in range(8):
                            tv = tailbuf[rr, pl.ds(16 * ii, 16)]
                            plsc.store_scatter(
                                stag, [splat_i(k), splat_i(rr),
                                       16 * ii + k16], tv)

                return z

            lax.fori_loop(0, NSEL, gfire, 0)

            def gdrain(k, z):
                bid = bid_of(k)

                @pl.when(jnp.logical_and(k < ncand, bid < LAST_BID))
                def _():
                    pltpu.make_async_copy(
                        out_ref.at[pl.ds(rg, 8), pl.ds(0, GBLK)],
                        stag.at[k], gsem).wait()

                return z

            lax.fori_loop(0, NSEL, gdrain, 0)

            def binit(k, bs):
                bid = bid_of(k)
                mx = jnp.where(k < ncand, blk_max(k, bid, i8),
                               jnp.float32(NEGINF))
                return tuple(
                    jnp.where(k16 + 16 * j4 == k, splat_f(mx), bs[j4])
                    for j4 in range(4))

            bs0 = tuple(splat_f(NEGINF) for _ in range(4))
            bs = lax.fori_loop(0, NSEL, binit, bs0)

            def step(t, bs):
                b0, b1, b2, b3 = bs
                gm = jnp.max(jnp.maximum(jnp.maximum(b0, b1),
                                         jnp.maximum(b2, b3)))
                jj = jnp.min(jnp.minimum(
                    jnp.minimum(jnp.where(b0 == gm, k16, 9999),
                                jnp.where(b1 == gm, k16 + 16, 9999)),
                    jnp.minimum(jnp.where(b2 == gm, k16 + 32, 9999),
                                jnp.where(b3 == gm, k16 + 48, 9999))))
                bid = bid_of(jj)
                cvs = []
                p = jnp.int32(9999)
                for ii in range(8):
                    cv = plsc.load_gather(stag, [splat_i(jj), splat_i(i8),
                                                 16 * ii + k16])
                    ok = bid * GBLK + 16 * ii + k16 < NPOI
                    cv = jnp.where(ok, cv, splat_f(NEGINF))
                    cvs.append(cv)
                    p = jnp.minimum(p, jnp.min(
                        jnp.where(cv == gm, 16 * ii + k16, 9999)))
                p = jnp.minimum(p, 127)
                nm = splat_f(NEGINF)
                for ii in range(8):
                    nm = jnp.maximum(nm, jnp.where(
                        16 * ii + k16 == p, splat_f(NEGINF), cvs[ii]))
                tv = splat_i(t)
                plsc.store_scatter(selidx, [tv],
                                   splat_i(jnp.minimum(bid * GBLK + p,
                                                       NPOI - 1)),
                                   mask=lane0)
                plsc.store_scatter(selloc, [tv], splat_i(jj * GBLK + p),
                                   mask=lane0)
                plsc.store_scatter(selval, [tv], splat_f(gm), mask=lane0)
                plsc.store_scatter(stag, [splat_i(jj), splat_i(i8),
                                          splat_i(p)],
                                   splat_f(NEGINF), mask=lane0)
                nms = jnp.max(nm)
                return tuple(
                    jnp.where(k16 + 16 * j4 == jj, splat_f(nms), bs[j4])
                    for j4 in range(4))

            lax.fori_loop(0, NSEL, step, bs)

            pltpu.async_copy(pr_hbm.at[selidx], ridv, ssem).wait()
            for v4 in range(4):
                loc = selloc[pl.ds(16 * v4, 16)]
                val = selval[pl.ds(16 * v4, 16)]
                rid = ridv[pl.ds(16 * v4, 16)]
                dv = plsc.load_gather(sc8, [splat_i(i8), rid])
                add = jnp.where(k16 + 16 * v4 < TOPM, dv, jnp.float32(0.0))
                plsc.store_scatter(
                    stag,
                    [lax.shift_right_logical(loc, 7), splat_i(i8),
                     jnp.bitwise_and(loc, 127)],
                    val + add)

            def wfire(k, z):
                bid = bid_of(k)

                @pl.when(jnp.logical_and(k < ncand, bid < LAST_BID))
                def _():
                    pltpu.async_copy(
                        stag.at[k],
                        out_ref.at[pl.ds(rg, 8), pl.ds(bid * GBLK, GBLK)],
                        osem)

                @pl.when(jnp.logical_and(k < ncand, bid >= LAST_BID))
                def _():
                    pltpu.async_copy(stag.at[k], tail_ref.at[pl.ds(rg, 8)],
                                     osem)

                return z

            lax.fori_loop(0, NSEL, wfire, 0)

            def wdrain(k, z):
                @pl.when(k < ncand)
                def _():
                    pltpu.make_async_copy(
                        stag.at[k], out_ref.at[pl.ds(rg, 8), pl.ds(0, GBLK)],
                        osem).wait()

                return z

            lax.fori_loop(0, NSEL, wdrain, 0)
            return zz

        lax.fori_loop(0, 8, row, 0)
        return carry

    lax.fori_loop(0, GPW, group, 0)


def _s2(bm, tau, scores, poi_region_id, out_ref, tail_ref):
    fn = pl.kernel(
        _apply_body,
        mesh=_mesh(),
        compiler_params=pltpu.CompilerParams(needs_layout_passes=False),
        out_type=(),
        scratch_types=[
            pltpu.VMEM((8, NBM), jnp.float32),
            pltpu.VMEM((8, 128), jnp.float32),
            pltpu.VMEM((8, 1024), jnp.float32),
            pltpu.VMEM((8, 128), jnp.float32),
            pltpu.VMEM((NSEL + 16,), jnp.int32),
            pltpu.VMEM((NSEL, 8, GBLK), jnp.float32),
            pltpu.VMEM((NSEL,), jnp.int32),
            pltpu.VMEM((NSEL,), jnp.int32),
            pltpu.VMEM((NSEL,), jnp.float32),
            pltpu.VMEM((NSEL,), jnp.int32),
            pltpu.SemaphoreType.DMA,
            pltpu.SemaphoreType.DMA,
            pltpu.SemaphoreType.DMA,
        ],
    )
    fn(bm, tau, scores, poi_region_id, out_ref, tail_ref)


def _tail_merge_body(src_ref, tv_ref, out_ref):
    del src_ref
    out_ref[...] = tv_ref[...]


def _k5(out_after, tailvals):
    return pl.pallas_call(
        _tail_merge_body,
        grid=(B // TB,),
        in_specs=[
            pl.BlockSpec(memory_space=pl.ANY),
            pl.BlockSpec((TB, 128), lambda i: (i, 0)),
        ],
        out_specs=pl.BlockSpec((TB, 128), lambda i: (i, LAST_BID)),
        out_shape=jax.ShapeDtypeStruct((B, NPOI), jnp.float32),
        input_output_aliases={0: 0},
    )(out_after, tailvals)


def kernel(pred_base, user_seq, user_seq_len, poi_region_id,
           fusion_batch_users_embs, fusion_pois_embs, region_emb, alpha):
    del fusion_batch_users_embs, fusion_pois_embs
    user_seq = user_seq.astype(jnp.int32)
    user_seq_len = user_seq_len.astype(jnp.int32)
    poi_region_id = poi_region_id.astype(jnp.int32)
    pred_base = pred_base.astype(jnp.float32)
    region_emb = region_emb.astype(jnp.float32)

    up = _s1(user_seq, user_seq_len, poi_region_id, region_emb)
    out0, bm = _k1(pred_base)
    alpha_arr = jnp.broadcast_to(
        jnp.asarray(alpha, jnp.float32).reshape(1, 1), (8, 128))
    tau, scores = _k2(alpha_arr, bm, up, region_emb)
    tail0 = jnp.pad(lax.slice(pred_base, (0, NPOI - LAST_W), (B, NPOI)),
                    ((0, 0), (0, GBLK - LAST_W)), constant_values=NEGINF)
    out_ref = jax.new_ref(out0)
    tail_ref = jax.new_ref(tail0)
    _s2(bm, tau, scores, poi_region_id, out_ref, tail_ref)
    out = _k5(jax.freeze(out_ref), jax.freeze(tail_ref))
    return out, up


# speedup vs baseline: 9.4541x; 1.0009x over previous
import jax
import jax.numpy as jnp
from jax import lax
from jax.experimental import pallas as pl
from jax.experimental.pallas import tpu as pltpu
from jax.experimental.pallas import tpu_sc as plsc

B = 1024
S = 200
NPOI = 100000
NREG = 1000
D = 64
RECENT_K = 10
TOPM = 50

TB = 8
WCOL = 16384
NCT = (NPOI + WCOL - 1) // WCOL
GBLK = 128
NBM = NCT * (WCOL // GBLK)
LAST_BID = (NPOI - 1) // GBLK
LAST_W = NPOI - LAST_BID * GBLK
NSEL = 64
NW = 32
RPW = B // NW
GPW = (B // 8) // NW
NEGINF = float("-inf")


def _copy_blockmax_body(x_ref, out_ref, bm_ref):
    j = pl.program_id(1)
    x = x_ref[...]
    out_ref[...] = x
    col = j * WCOL + lax.broadcasted_iota(jnp.int32, (TB, WCOL), 1)
    xm = jnp.where(col < NPOI, x, NEGINF)
    bm_ref[...] = jnp.max(xm.reshape(TB, WCOL // GBLK, GBLK), axis=2)


@jax.jit
def _k1(pred):
    return pl.pallas_call(
        _copy_blockmax_body,
        grid=(B // TB, NCT),
        in_specs=[pl.BlockSpec((TB, WCOL), lambda i, j: (i, j))],
        out_specs=[
            pl.BlockSpec((TB, WCOL), lambda i, j: (i, j)),
            pl.BlockSpec((TB, WCOL // GBLK), lambda i, j: (i, j)),
        ],
        out_shape=[
            jax.ShapeDtypeStruct((B, NPOI), jnp.float32),
            jax.ShapeDtypeStruct((B, NBM), jnp.float32),
        ],
    )(pred)


def _thresh_body(alpha_ref, bm_ref, up_ref, re_ref, tau_ref, sc_ref):
    bmv = bm_ref[...]
    bits = lax.bitcast_convert_type(bmv, jnp.int32)
    key = bits ^ jnp.where(bits < 0, jnp.int32(0x7FFFFFFF), jnp.int32(0))

    t0 = jnp.full((TB, 1), jnp.int32(-(2**31)))

    def it(k, t):
        c = t + lax.shift_left(jnp.int32(1), jnp.int32(31) - k)
        cnt = jnp.sum((key >= c).astype(jnp.int32), axis=1, keepdims=True)
        return jnp.where(cnt >= TOPM, c, t)

    t = lax.fori_loop(0, 32, it, t0)
    fb = t ^ jnp.where(t < 0, jnp.int32(0x7FFFFFFF), jnp.int32(0))
    tau = lax.bitcast_convert_type(fb, jnp.float32)
    tau_ref[...] = jnp.broadcast_to(tau, (TB, 128))

    up = up_ref[...]
    sc = lax.dot_general(up, re_ref[...], (((1,), (1,)), ((), ())),
                         preferred_element_type=jnp.float32)
    sc = sc * alpha_ref[...][0:TB, 0:1]
    sc_ref[...] = jnp.concatenate(
        [sc, jnp.zeros((TB, 1024 - NREG), jnp.float32)], axis=1)


@jax.jit
def _k2(alpha_arr, bm, up, re):
    return pl.pallas_call(
        _thresh_body,
        grid=(B // TB,),
        in_specs=[
            pl.BlockSpec((8, 128), lambda i: (0, 0)),
            pl.BlockSpec((TB, NBM), lambda i: (i, 0)),
            pl.BlockSpec((TB, D), lambda i: (i, 0)),
            pl.BlockSpec((NREG, D), lambda i: (0, 0)),
        ],
        out_specs=[
            pl.BlockSpec((TB, 128), lambda i: (i, 0)),
            pl.BlockSpec((TB, 1024), lambda i: (i, 0)),
        ],
        out_shape=[
            jax.ShapeDtypeStruct((B, 128), jnp.float32),
            jax.ShapeDtypeStruct((B, 1024), jnp.float32),
        ],
    )(alpha_arr, bm, up, re)


def _mesh():
    return plsc.VectorSubcoreMesh(core_axis_name="c", subcore_axis_name="s")


def _user_pref_body(seq_hbm, len_hbm, pr_hbm, re_hbm, up_hbm,
                    seqv, lenv, ridv, embv, outv, sem0, sem1):
    cc = lax.axis_index("c")
    ss = lax.axis_index("s")
    wid = ss * 2 + cc
    r0 = wid * RPW
    pltpu.sync_copy(seq_hbm.at[pl.ds(r0, RPW)], seqv)
    pltpu.sync_copy(len_hbm.at[pl.ds(r0, RPW)], lenv)
    k16 = lax.iota(jnp.int32, 16)

    def row(i, carry):
        iv = jnp.full((16,), i, jnp.int32)
        ln = jnp.max(plsc.load_gather(lenv, [iv]))
        start = jnp.maximum(ln - RECENT_K, 0)
        cnt = ln - start
        pos = jnp.minimum(start + k16, S - 1)
        pid = plsc.load_gather(seqv, [iv, pos])
        pltpu.async_copy(pr_hbm.at[pid], ridv, sem0).wait()
        pltpu.async_copy(re_hbm.at[ridv], embv, sem1).wait()
        w = jnp.where(k16 < cnt, jnp.float32(1.0), jnp.float32(0.0))
        inv = jnp.float32(1.0)
        for n in range(2, RECENT_K + 1):
            inv = jnp.where(cnt == n, jnp.float32(1.0 / n), inv)
        for jj in range(D // 16):
            acc = jnp.zeros((16,), jnp.float32)
            for tt in range(16):
                wt = jnp.max(jnp.where(k16 == tt, w, jnp.float32(0.0)))
                acc = acc + embv[tt, pl.ds(16 * jj, 16)] * wt
            plsc.store_scatter(outv, [iv, 16 * jj + k16], acc * inv)
        return carry

    lax.fori_loop(0, RPW, row, 0)
    pltpu.sync_copy(outv, up_hbm.at[pl.ds(r0, RPW)])


@jax.jit
def _s1(user_seq, user_seq_len, poi_region_id, region_emb):
    region_emb = jnp.pad(region_emb, ((0, 0), (0, 128 - D)))
    fn = pl.kernel(
        _user_pref_body,
        mesh=_mesh(),
        compiler_params=pltpu.CompilerParams(needs_layout_passes=False),
        out_type=jax.ShapeDtypeStruct((B, D), jnp.float32),
        scratch_types=[
            pltpu.VMEM((RPW, S), jnp.int32),
            pltpu.VMEM((RPW,), jnp.int32),
            pltpu.VMEM((16,), jnp.int32),
            pltpu.VMEM((16, 128), jnp.float32),
            pltpu.VMEM((RPW, D), jnp.float32),
            pltpu.SemaphoreType.DMA,
            pltpu.SemaphoreType.DMA,
        ],
    )
    return fn(user_seq, user_seq_len, poi_region_id, region_emb)


def _apply_body(bm_hbm, tau_hbm, sc_hbm, pr_hbm, out_ref, tail_ref,
                bm8, tau8, sc8, tailbuf, idsv, stag, selidx, selloc,
                selval, ridv, gsem, ssem, osem):
    cc = lax.axis_index("c")
    ss = lax.axis_index("s")
    wid = ss * 2 + cc
    k16 = lax.iota(jnp.int32, 16)
    lane0 = k16 == 0

    def splat_i(x):
        return jnp.full((16,), x, jnp.int32)

    def splat_f(x):
        return jnp.full((16,), x, jnp.float32)

    def bid_of(k):
        return jnp.max(plsc.load_gather(idsv, [splat_i(k)]))

    def blk_max(k, bid, i8):
        m = splat_f(NEGINF)
        for ii in range(8):
            cv = plsc.load_gather(stag, [splat_i(k), splat_i(i8),
                                         16 * ii + k16])
            ok = bid * GBLK + 16 * ii + k16 < NPOI
            m = jnp.maximum(m, jnp.where(ok, cv, splat_f(NEGINF)))
        return jnp.max(m)

    def group(gl, carry):
        g = wid * GPW + gl
        rg = g * 8
        pltpu.sync_copy(bm_hbm.at[pl.ds(rg, 8)], bm8)
        pltpu.sync_copy(tau_hbm.at[pl.ds(rg, 8)], tau8)
        pltpu.sync_copy(sc_hbm.at[pl.ds(rg, 8)], sc8)

        def row(i8, zz):
            tau = jnp.max(plsc.load_gather(tau8, [splat_i(i8), k16]))

            for j5 in range(5):
                idsv[pl.ds(16 * j5, 16)] = k16 + 16 * j5
            cnt = jnp.int32(0)
            for j in range(NBM // 16):
                v = plsc.load_gather(bm8, [splat_i(i8), 16 * j + k16])
                m = v >= tau
                c_now = cnt

                @pl.when(c_now < NSEL)
                def _():
                    plsc.store_compressed(idsv.at[pl.ds(c_now, 16)],
                                          k16 + 16 * j, mask=m)

                cnt = cnt + jnp.max(plsc.all_reduce_population_count(m))

            ncand = jnp.minimum(cnt, NSEL)

            def gfire(k, z):
                bid = bid_of(k)

                @pl.when(jnp.logical_and(k < ncand, bid < LAST_BID))
                def _():
                    pltpu.async_copy(
                        out_ref.at[pl.ds(rg, 8), pl.ds(bid * GBLK, GBLK)],
                        stag.at[k], gsem)

                @pl.when(jnp.logical_and(k < ncand, bid >= LAST_BID))
                def _():
                    pltpu.sync_copy(tail_ref.at[pl.ds(rg, 8)], tailbuf)
                    for rr in range(8):
                        for ii in range(8):
                            tv = tailbuf[rr, pl.ds(16 * ii, 16)]
                            plsc.store_scatter(
                                stag, [splat_i(k), splat_i(rr),
                                       16 * ii + k16], tv)

                return z

            lax.fori_loop(0, NSEL, gfire, 0)

            def gdrain(k, z):
                bid = bid_of(k)

                @pl.when(jnp.logical_and(k < ncand, bid < LAST_BID))
                def _():
                    pltpu.make_async_copy(
                        out_ref.at[pl.ds(rg, 8), pl.ds(0, GBLK)],
                        stag.at[k], gsem).wait()

                return z

            lax.fori_loop(0, NSEL, gdrain, 0)

            def binit(k, bs):
                bid = bid_of(k)
                mx = jnp.where(k < ncand, blk_max(k, bid, i8),
                               jnp.float32(NEGINF))
                return tuple(
                    jnp.where(k16 + 16 * j4 == k, splat_f(mx), bs[j4])
                    for j4 in range(4))

            bs0 = tuple(splat_f(NEGINF) for _ in range(4))
            bs = lax.fori_loop(0, NSEL, binit, bs0)

            def step(t, bs):
                b0, b1, b2, b3 = bs
                gm = jnp.max(jnp.maximum(jnp.maximum(b0, b1),
                                         jnp.maximum(b2, b3)))
                jj = jnp.min(jnp.minimum(
                    jnp.minimum(jnp.where(b0 == gm, k16, 9999),
                                jnp.where(b1 == gm, k16 + 16, 9999)),
                    jnp.minimum(jnp.where(b2 == gm, k16 + 32, 9999),
                                jnp.where(b3 == gm, k16 + 48, 9999))))
                bid = bid_of(jj)
                cvs = []
                p = jnp.int32(9999)
                for ii in range(8):
                    cv = plsc.load_gather(stag, [splat_i(jj), splat_i(i8),
                                                 16 * ii + k16])
                    ok = bid * GBLK + 16 * ii + k16 < NPOI
                    cv = jnp.where(ok, cv, splat_f(NEGINF))
                    cvs.append(cv)
                    p = jnp.minimum(p, jnp.min(
                        jnp.where(cv == gm, 16 * ii + k16, 9999)))
                p = jnp.minimum(p, 127)
                nm = splat_f(NEGINF)
                for ii in range(8):
                    nm = jnp.maximum(nm, jnp.where(
                        16 * ii + k16 == p, splat_f(NEGINF), cvs[ii]))
                tv = splat_i(t)
                plsc.store_scatter(selidx, [tv],
                                   splat_i(jnp.minimum(bid * GBLK + p,
                                                       NPOI - 1)),
                                   mask=lane0)
                plsc.store_scatter(selloc, [tv], splat_i(jj * GBLK + p),
                                   mask=lane0)
                plsc.store_scatter(selval, [tv], splat_f(gm), mask=lane0)
                plsc.store_scatter(stag, [splat_i(jj), splat_i(i8),
                                          splat_i(p)],
                                   splat_f(NEGINF), mask=lane0)
                nms = jnp.max(nm)
                return tuple(
                    jnp.where(k16 + 16 * j4 == jj, splat_f(nms), bs[j4])
                    for j4 in range(4))

            lax.fori_loop(0, NSEL, step, bs)

            pltpu.async_copy(pr_hbm.at[selidx], ridv, ssem).wait()
            for v4 in range(4):
                loc = selloc[pl.ds(16 * v4, 16)]
                val = selval[pl.ds(16 * v4, 16)]
                rid = ridv[pl.ds(16 * v4, 16)]
                dv = plsc.load_gather(sc8, [splat_i(i8), rid])
                add = jnp.where(k16 + 16 * v4 < TOPM, dv, jnp.float32(0.0))
                plsc.store_scatter(
                    stag,
                    [lax.shift_right_logical(loc, 7), splat_i(i8),
                     jnp.bitwise_and(loc, 127)],
                    val + add)

            def wfire(k, z):
                bid = bid_of(k)

                @pl.when(jnp.logical_and(k < ncand, bid < LAST_BID))
                def _():
                    pltpu.async_copy(
                        stag.at[k],
                        out_ref.at[pl.ds(rg, 8), pl.ds(bid * GBLK, GBLK)],
                        osem)

                @pl.when(jnp.logical_and(k < ncand, bid >= LAST_BID))
                def _():
                    pltpu.async_copy(stag.at[k], tail_ref.at[pl.ds(rg, 8)],
                                     osem)

                return z

            lax.fori_loop(0, NSEL, wfire, 0)

            def wdrain(k, z):
                @pl.when(k < ncand)
                def _():
                    pltpu.make_async_copy(
                        stag.at[k], out_ref.at[pl.ds(rg, 8), pl.ds(0, GBLK)],
                        osem).wait()

                return z

            lax.fori_loop(0, NSEL, wdrain, 0)
            return zz

        lax.fori_loop(0, 8, row, 0)
        return carry

    lax.fori_loop(0, GPW, group, 0)


def _s2(bm, tau, scores, poi_region_id, out_ref, tail_ref):
    fn = pl.kernel(
        _apply_body,
        mesh=_mesh(),
        compiler_params=pltpu.CompilerParams(needs_layout_passes=False),
        out_type=(),
        scratch_types=[
            pltpu.VMEM((8, NBM), jnp.float32),
            pltpu.VMEM((8, 128), jnp.float32),
            pltpu.VMEM((8, 1024), jnp.float32),
            pltpu.VMEM((8, 128), jnp.float32),
            pltpu.VMEM((NSEL + 16,), jnp.int32),
            pltpu.VMEM((NSEL, 8, GBLK), jnp.float32),
            pltpu.VMEM((NSEL,), jnp.int32),
            pltpu.VMEM((NSEL,), jnp.int32),
            pltpu.VMEM((NSEL,), jnp.float32),
            pltpu.VMEM((NSEL,), jnp.int32),
            pltpu.SemaphoreType.DMA,
            pltpu.SemaphoreType.DMA,
            pltpu.SemaphoreType.DMA,
        ],
    )
    fn(bm, tau, scores, poi_region_id, out_ref, tail_ref)


def _tail_merge_body(src_ref, tv_ref, out_ref):
    del src_ref
    out_ref[...] = tv_ref[...]


def _k5(out_after, tailvals):
    return pl.pallas_call(
        _tail_merge_body,
        grid=(B // TB,),
        in_specs=[
            pl.BlockSpec(memory_space=pl.ANY),
            pl.BlockSpec((TB, 128), lambda i: (i, 0)),
        ],
        out_specs=pl.BlockSpec((TB, 128), lambda i: (i, LAST_BID)),
        out_shape=jax.ShapeDtypeStruct((B, NPOI), jnp.float32),
        input_output_aliases={0: 0},
    )(out_after, tailvals)


def kernel(pred_base, user_seq, user_seq_len, poi_region_id,
           fusion_batch_users_embs, fusion_pois_embs, region_emb, alpha):
    del fusion_batch_users_embs, fusion_pois_embs
    user_seq = user_seq.astype(jnp.int32)
    user_seq_len = user_seq_len.astype(jnp.int32)
    poi_region_id = poi_region_id.astype(jnp.int32)
    pred_base = pred_base.astype(jnp.float32)
    region_emb = region_emb.astype(jnp.float32)

    up = _s1(user_seq, user_seq_len, poi_region_id, region_emb)
    out0, bm = _k1(pred_base)
    alpha_arr = jnp.broadcast_to(
        jnp.asarray(alpha, jnp.float32).reshape(1, 1), (8, 128))
    tau, scores = _k2(alpha_arr, bm, up, region_emb)
    tail0 = jnp.pad(lax.slice(pred_base, (0, NPOI - LAST_W), (B, NPOI)),
                    ((0, 0), (0, GBLK - LAST_W)), constant_values=NEGINF)
    out_ref = jax.new_ref(out0)
    tail_ref = jax.new_ref(tail0)
    _s2(bm, tau, scores, poi_region_id, out_ref, tail_ref)
    out = _k5(jax.freeze(out_ref), jax.freeze(tail_ref))
    return out, up
